# Initial kernel scaffold; baseline (speedup 1.0000x reference)
import jax
import jax.numpy as jnp
from jax.experimental import pallas as pl


def kernel(x, edge_index, edge_weight, W1, b1, W2, b2):
    raise NotImplementedError("write your pallas kernel here")


# speedup vs baseline: 3.0367x; 3.0367x over previous
import jax
import jax.numpy as jnp
from jax import lax
from jax.experimental import pallas as pl
from jax.experimental.pallas import tpu as pltpu
from jax.experimental.pallas import tpu_sc as plsc

N = 10000
D = 128
E = 320000

NC = 2
NS = 16
NW = NC * NS

CHUNK = 128
NCHUNKS = 80
EPW = CHUNK * NCHUNKS
EP = EPW * NW

_LANE_DNUMS = lax.GatherDimensionNumbers(
    offset_dims=(), collapsed_slice_dims=(0,), start_index_map=(0,))


def _lane_broadcast(vec, j):
    idx = jnp.full((16, 1), j, dtype=jnp.int32)
    return lax.gather(vec, idx, _LANE_DNUMS, (1,),
                      mode=lax.GatherScatterMode.PROMISE_IN_BOUNDS)


def _agg_body(x_hbm, src_hbm, dst_hbm, w_hbm, z_hbm, out_hbm,
              src_v, w_v, dst_v, rows_v, acc_sh, sem):
    c = lax.axis_index("c")
    s = lax.axis_index("s")
    wid = s * NC + c

    @pl.when(s < 15)
    def _():
        pltpu.sync_copy(z_hbm.at[pl.ds(0, 624)],
                        acc_sh.at[pl.ds(s * 624, 624)])

    @pl.when(s == 15)
    def _():
        pltpu.sync_copy(z_hbm, acc_sh.at[pl.ds(15 * 624, 640)])

    pltpu.sync_copy(src_hbm.at[wid], src_v)
    pltpu.sync_copy(dst_hbm.at[wid], dst_v)
    pltpu.sync_copy(w_hbm.at[wid], w_v)
    plsc.subcore_barrier()

    def chunk_body(k, carry):
        pltpu.async_copy(x_hbm.at[src_v.at[k]], rows_v, sem).wait()

        def mul_group(g, c2):
            wv = w_v[k, pl.ds(g * 16, 16)]
            for j in range(16):
                wb = _lane_broadcast(wv, j)
                e = g * 16 + j
                for d in range(8):
                    sl = pl.ds(d * 16, 16)
                    rows_v[e, sl] = rows_v[e, sl] * wb
            return c2
        lax.fori_loop(0, CHUNK // 16, mul_group, 0)

        pltpu.sync_copy(rows_v, acc_sh.at[dst_v.at[k]], add=True)
        return carry

    lax.fori_loop(0, NCHUNKS, chunk_body, 0)

    plsc.subcore_barrier()

    @pl.when(s < 15)
    def _():
        pltpu.sync_copy(acc_sh.at[pl.ds(s * 624, 624)],
                        out_hbm.at[c, pl.ds(s * 624, 624)])

    @pl.when(s == 15)
    def _():
        pltpu.sync_copy(acc_sh.at[pl.ds(15 * 624, 640)],
                        out_hbm.at[c, pl.ds(15 * 624, 640)])


_agg_call = pl.kernel(
    _agg_body,
    out_type=jax.ShapeDtypeStruct((NC, N, D), jnp.float32),
    mesh=plsc.VectorSubcoreMesh(core_axis_name="c", subcore_axis_name="s"),
    scratch_types=[
        pltpu.VMEM((NCHUNKS, CHUNK), jnp.int32),
        pltpu.VMEM((NCHUNKS, CHUNK), jnp.float32),
        pltpu.VMEM((NCHUNKS, CHUNK), jnp.int32),
        pltpu.VMEM((CHUNK, D), jnp.float32),
        pltpu.VMEM_SHARED((N, D), jnp.float32),
        pltpu.SemaphoreType.DMA,
    ],
)


def _dense(p, W, b, relu):
    def body(p_ref, w_ref, b_ref, o_ref):
        acc = p_ref[0] + p_ref[1]
        r = jnp.dot(acc, w_ref[...], preferred_element_type=jnp.float32,
                    precision=lax.Precision.HIGHEST) + b_ref[...]
        o_ref[...] = jnp.maximum(r, 0.0) if relu else r

    R = 1000
    return pl.pallas_call(
        body,
        grid=(N // R,),
        in_specs=[
            pl.BlockSpec((2, R, D), lambda i: (0, i, 0)),
            pl.BlockSpec((D, D), lambda i: (0, 0)),
            pl.BlockSpec((1, D), lambda i: (0, 0)),
        ],
        out_specs=pl.BlockSpec((R, D), lambda i: (i, 0)),
        out_shape=jax.ShapeDtypeStruct((N, D), jnp.float32),
    )(p, W, b.reshape(1, D))


def kernel(x, edge_index, edge_weight, W1, b1, W2, b2):
    src = edge_index[0].astype(jnp.int32)
    dst = edge_index[1].astype(jnp.int32)
    w = edge_weight.astype(jnp.float32)
    pad = EP - E
    src_p = jnp.pad(src, (0, pad)).reshape(NW, NCHUNKS, CHUNK)
    dst_p = jnp.pad(dst, (0, pad)).reshape(NW, NCHUNKS, CHUNK)
    w_p = jnp.pad(w, (0, pad)).reshape(NW, NCHUNKS, CHUNK)
    zeros = jnp.zeros((640, D), jnp.float32)

    p1 = _agg_call(x, src_p, dst_p, w_p, zeros)
    h = _dense(p1, W1, b1, relu=True)
    p2 = _agg_call(h, src_p, dst_p, w_p, zeros)
    return _dense(p2, W2, b2, relu=False)
